# baseline (speedup 1.0000x reference)
import jax
import jax.numpy as jnp
from jax.experimental import pallas as pl
from jax.experimental.pallas import tpu as pltpu

C = 2048
E = 131072
D = 32
H = 32
K = 2
MSG_H = 64
MSG_O = 32
TAU = 0.1


def _mlp_body(x_ref, w1_ref, b1_ref, w2_ref, b2_ref, g_ref, be_ref, o_ref):
    x = x_ref[...]
    x = jnp.maximum(x @ w1_ref[...] + b1_ref[...], 0.0)
    x = jnp.maximum(x @ w2_ref[...] + b2_ref[...], 0.0)
    mean = jnp.mean(x, axis=0, keepdims=True)
    var = jnp.mean((x - mean) ** 2, axis=0, keepdims=True)
    x = (x - mean) * jax.lax.rsqrt(var + 1e-5)
    o_ref[...] = x * g_ref[...] + be_ref[...]


def _pallas_mlp(x, p, name):
    din = x.shape[1]
    dh = p[name + '_w1'].shape[1]
    dout = p[name + '_w2'].shape[1]
    return pl.pallas_call(
        _mlp_body,
        out_shape=jax.ShapeDtypeStruct((x.shape[0], dout), jnp.float32),
    )(x, p[name + '_w1'], p[name + '_b1'].reshape(1, dh),
      p[name + '_w2'], p[name + '_b2'].reshape(1, dout),
      p[name + '_g'].reshape(1, dout), p[name + '_be'].reshape(1, dout))


def _node2edge(x, send_idx, rec_idx):
    senders = jnp.take(x, send_idx, axis=0)
    receivers = jnp.take(x, rec_idx, axis=0)
    return jnp.concatenate([senders, receivers], axis=-1)


def kernel(data, params, send_idx, rec_idx):
    p = params
    def _xla_mlp(x, name):
        x = jax.nn.relu(x @ p[name + '_w1'] + p[name + '_b1'])
        x = jax.nn.relu(x @ p[name + '_w2'] + p[name + '_b2'])
        mean = jnp.mean(x, axis=0, keepdims=True)
        var = jnp.var(x, axis=0, keepdims=True)
        x = (x - mean) / jnp.sqrt(var + 1e-5)
        return x * p[name + '_g'] + p[name + '_be']

    x = _pallas_mlp(data, p, 'enc1')
    x = _node2edge(x, send_idx, rec_idx)
    x = _xla_mlp(x, 'enc2')
    x_skip = x
    x = jax.ops.segment_sum(x, rec_idx, num_segments=C) / C
    x = _pallas_mlp(x, p, 'enc3')
    x = _node2edge(x, send_idx, rec_idx)
    x = jnp.concatenate([x, x_skip], axis=-1)
    x = _xla_mlp(x, 'enc4')
    logits = x @ p['fc_out_w'] + p['fc_out_b']
    u = jax.random.uniform(jax.random.key(42), logits.shape, minval=1e-6, maxval=1.0 - 1e-6)
    g = -jnp.log(-jnp.log(u))
    edges = jax.nn.softmax((logits + g) / TAU, axis=-1)
    prob = jax.nn.softmax(logits, axis=-1)
    pre_msg = _node2edge(data, send_idx, rec_idx)
    all_msgs = jnp.zeros((pre_msg.shape[0], MSG_O), jnp.float32)
    for i in range(K):
        m = jax.nn.relu(pre_msg @ p['msg1_%d_w' % i] + p['msg1_%d_b' % i])
        m = jax.nn.relu(m @ p['msg2_%d_w' % i] + p['msg2_%d_b' % i])
        all_msgs = all_msgs + m * edges[:, i:i + 1]
    agg = jax.ops.segment_sum(all_msgs, rec_idx, num_segments=C) / C
    pred = jax.nn.relu(agg @ p['out1_w'] + p['out1_b'])
    output = pred @ p['out2_w'] + p['out2_b']
    graphs = jnp.zeros((K, C, C), jnp.float32)
    for k in range(K):
        graphs = graphs.at[k, send_idx, rec_idx].set(edges[:, k])
    return graphs, output, prob


# speedup vs baseline: 1.1006x; 1.1006x over previous
import dataclasses
import functools

import jax
import jax.numpy as jnp
from jax import lax
from jax.experimental import pallas as pl
from jax.experimental.pallas import tpu as pltpu
from jax.experimental.pallas import tpu_sc as plsc

C = 2048
E = 131072
D = 32
H = 32
K = 2
MSG_H = 64
MSG_O = 32
TAU = 0.1


_NC, _NS, _L = 2, 16, 16
_NW = _NC * _NS
_REG = 65536
_NREG = (C * C) // _REG
_ROUNDS = _NREG // _NW
_CH = 2048
_NCH = E // _CH


def _sc_compiler_params():
    cp = pltpu.CompilerParams()
    if "needs_layout_passes" in pltpu.CompilerParams.__dataclass_fields__:
        cp = dataclasses.replace(cp, needs_layout_passes=False)
    return cp


def _scatter_graphs(lin2d, v0, v1):
    mesh = plsc.VectorSubcoreMesh(core_axis_name="c", subcore_axis_name="s")

    @functools.partial(
        pl.kernel,
        out_type=jax.ShapeDtypeStruct((K, C * C), jnp.float32),
        mesh=mesh,
        scratch_types=[pltpu.VMEM((_REG,), jnp.float32)],
        compiler_params=_sc_compiler_params(),
    )
    def k(lin_hbm, v0_hbm, v1_hbm, out_hbm, reg):
        wid = lax.axis_index("s") * _NC + lax.axis_index("c")
        for kk in range(K):
            v_hbm = v0_hbm if kk == 0 else v1_hbm
            for rnd in range(_ROUNDS):
                base = (rnd * _NW + wid) * _REG

                @pl.loop(0, _REG // _L)
                def _(i):
                    reg[pl.ds(i * _L, _L)] = jnp.zeros((_L,), jnp.float32)

                def body(lin_v, val_v, base=base):
                    @pl.loop(0, _CH // _L)
                    def _(j):
                        idx = lin_v[0, pl.ds(j * _L, _L)]
                        off = idx - base
                        msk = (off >= 0) & (off < _REG)
                        offc = jnp.where(msk, off, 0)
                        val = val_v[0, pl.ds(j * _L, _L)]
                        plsc.store_scatter(reg, [offc], val, mask=msk)

                pltpu.emit_pipeline(
                    body,
                    grid=(_NCH,),
                    in_specs=[
                        pl.BlockSpec((1, _CH), lambda i: (i, 0)),
                        pl.BlockSpec((1, _CH), lambda i: (i, 0)),
                    ],
                )(lin_hbm, v_hbm)
                pltpu.sync_copy(reg, out_hbm.at[kk, pl.ds(base, _REG)])

    return k(lin2d, v0, v1)


def _mlp_body(x_ref, w1_ref, b1_ref, w2_ref, b2_ref, g_ref, be_ref, o_ref):
    x = x_ref[...]
    x = jnp.maximum(x @ w1_ref[...] + b1_ref[...], 0.0)
    x = jnp.maximum(x @ w2_ref[...] + b2_ref[...], 0.0)
    mean = jnp.mean(x, axis=0, keepdims=True)
    var = jnp.mean((x - mean) ** 2, axis=0, keepdims=True)
    x = (x - mean) * jax.lax.rsqrt(var + 1e-5)
    o_ref[...] = x * g_ref[...] + be_ref[...]


def _pallas_mlp(x, p, name):
    din = x.shape[1]
    dh = p[name + '_w1'].shape[1]
    dout = p[name + '_w2'].shape[1]
    return pl.pallas_call(
        _mlp_body,
        out_shape=jax.ShapeDtypeStruct((x.shape[0], dout), jnp.float32),
    )(x, p[name + '_w1'], p[name + '_b1'].reshape(1, dh),
      p[name + '_w2'], p[name + '_b2'].reshape(1, dout),
      p[name + '_g'].reshape(1, dout), p[name + '_be'].reshape(1, dout))


def _node2edge(x, send_idx, rec_idx):
    senders = jnp.take(x, send_idx, axis=0)
    receivers = jnp.take(x, rec_idx, axis=0)
    return jnp.concatenate([senders, receivers], axis=-1)


def kernel(data, params, send_idx, rec_idx):
    p = params
    def _xla_mlp(x, name):
        x = jax.nn.relu(x @ p[name + '_w1'] + p[name + '_b1'])
        x = jax.nn.relu(x @ p[name + '_w2'] + p[name + '_b2'])
        mean = jnp.mean(x, axis=0, keepdims=True)
        var = jnp.var(x, axis=0, keepdims=True)
        x = (x - mean) / jnp.sqrt(var + 1e-5)
        return x * p[name + '_g'] + p[name + '_be']

    x = _pallas_mlp(data, p, 'enc1')
    x = _node2edge(x, send_idx, rec_idx)
    x = _xla_mlp(x, 'enc2')
    x_skip = x
    x = jax.ops.segment_sum(x, rec_idx, num_segments=C) / C
    x = _pallas_mlp(x, p, 'enc3')
    x = _node2edge(x, send_idx, rec_idx)
    x = jnp.concatenate([x, x_skip], axis=-1)
    x = _xla_mlp(x, 'enc4')
    logits = x @ p['fc_out_w'] + p['fc_out_b']
    u = jax.random.uniform(jax.random.key(42), logits.shape, minval=1e-6, maxval=1.0 - 1e-6)
    g = -jnp.log(-jnp.log(u))
    edges = jax.nn.softmax((logits + g) / TAU, axis=-1)
    prob = jax.nn.softmax(logits, axis=-1)
    pre_msg = _node2edge(data, send_idx, rec_idx)
    all_msgs = jnp.zeros((pre_msg.shape[0], MSG_O), jnp.float32)
    for i in range(K):
        m = jax.nn.relu(pre_msg @ p['msg1_%d_w' % i] + p['msg1_%d_b' % i])
        m = jax.nn.relu(m @ p['msg2_%d_w' % i] + p['msg2_%d_b' % i])
        all_msgs = all_msgs + m * edges[:, i:i + 1]
    agg = jax.ops.segment_sum(all_msgs, rec_idx, num_segments=C) / C
    pred = jax.nn.relu(agg @ p['out1_w'] + p['out1_b'])
    output = pred @ p['out2_w'] + p['out2_b']
    lin2d = (send_idx * C + rec_idx).reshape(_NCH, _CH)
    v0 = edges[:, 0].reshape(_NCH, _CH)
    v1 = edges[:, 1].reshape(_NCH, _CH)
    graphs = _scatter_graphs(lin2d, v0, v1).reshape(K, C, C)
    return graphs, output, prob
